# baseline (speedup 1.0000x reference)
import functools

import jax
import jax.numpy as jnp
from jax import lax
from jax.experimental import pallas as pl
from jax.experimental.pallas import tpu as pltpu
from jax.experimental.pallas import tpu_sc as plsc

_N = 10000
_NPAD = 10240
_E = 320000
_D = 128
_DO = 32
_EPS = 1e-5

_NC = 2
_NS = 16
_NW = _NC * _NS
_EPW = _E // _NW
_EPT = _E // _NS
_CH = 80
_NCHUNK = _EPW // _CH
_RPT = _NPAD // _NS

_mesh = plsc.VectorSubcoreMesh(
    core_axis_name="c", subcore_axis_name="s", num_cores=_NC, num_subcores=_NS)

_f32 = jnp.float32
_i32 = jnp.int32


def _rsqrt16(d):
    x = 0.5 * (d + 1.0)
    for _ in range(15):
        x = 0.5 * (x + d / x)
    return 1.0 / x


def _lane_splat(v16, j):
    return jax.lax.gather(
        v16, jnp.full((16, 1), j, _i32),
        jax.lax.GatherDimensionNumbers(
            offset_dims=(), collapsed_slice_dims=(0,), start_index_map=(0,)),
        slice_sizes=(1,),
        mode=jax.lax.GatherScatterMode.PROMISE_IN_BOUNDS)


def _k0_body(src_hbm, dst_hbm, ew_hbm, rin_hbm, w_hbm,
             eidx_res, ew_res, wbuf, ldeg, tbuf, dacc, rbuf, r_full, sh_sp):
    c = lax.axis_index("c")
    s = lax.axis_index("s")

    def _z(j, _):
        ldeg[pl.ds(j * 16, 16)] = jnp.zeros((16,), _f32)
        return 0
    lax.fori_loop(0, _NPAD // 16, _z, 0)

    @pl.when(c == 0)
    def _():
        pltpu.sync_copy(src_hbm.at[pl.ds(s * _EPT, _EPT)], eidx_res)

    @pl.when(c == 1)
    def _():
        pltpu.sync_copy(dst_hbm.at[pl.ds(s * _EPT, _EPT)], eidx_res)

    def _hist(i, _):
        idx16 = eidx_res[pl.ds(i * 16, 16)]
        plsc.addupdate_scatter(ldeg, [idx16], jnp.ones((16,), _f32))
        return 0
    lax.fori_loop(0, _EPT // 16, _hist, 0)

    pltpu.sync_copy(ldeg, sh_sp.at[pl.ds(s * _NPAD, _NPAD)])
    plsc.subcore_barrier()

    def _zt(j, _):
        dacc[pl.ds(j * 16, 16)] = jnp.zeros((16,), _f32)
        return 0
    lax.fori_loop(0, _RPT // 16, _zt, 0)

    def _red(r, _):
        pltpu.sync_copy(sh_sp.at[pl.ds(r * _NPAD + s * _RPT, _RPT)], tbuf)

        def _add(j, _):
            dacc[pl.ds(j * 16, 16)] = (dacc[pl.ds(j * 16, 16)]
                                       + tbuf[pl.ds(j * 16, 16)])
            return 0
        lax.fori_loop(0, _RPT // 16, _add, 0)
        return 0
    lax.fori_loop(0, _NS, _red, 0)

    def _r(j, _):
        d = jnp.maximum(dacc[pl.ds(j * 16, 16)], 1.0)
        rbuf[pl.ds(j * 16, 16)] = _rsqrt16(d)
        return 0
    lax.fori_loop(0, _RPT // 16, _r, 0)

    pltpu.sync_copy(rbuf, sh_sp.at[pl.ds(_NS * _NPAD + s * _RPT, _RPT)])

    @pl.when(c == 1)
    def _():
        pltpu.sync_copy(rbuf, rin_hbm.at[pl.ds(s * _RPT, _RPT)])

    plsc.subcore_barrier()

    @pl.when(c == 0)
    def _():
        pltpu.sync_copy(sh_sp.at[pl.ds(_NS * _NPAD, _NPAD)], r_full)
        pltpu.sync_copy(ew_hbm.at[pl.ds(s * _EPT, _EPT)], ew_res)

        def _w(i, _):
            idx16 = eidx_res[pl.ds(i * 16, 16)]
            r16 = plsc.load_gather(r_full, [idx16])
            wbuf[pl.ds(i * 16, 16)] = ew_res[pl.ds(i * 16, 16)] * r16
            return 0
        lax.fori_loop(0, _EPT // 16, _w, 0)
        pltpu.sync_copy(wbuf, w_hbm.at[pl.ds(s * _EPT, _EPT)])


_k0 = functools.partial(
    pl.kernel,
    out_type=(
        jax.ShapeDtypeStruct((_NPAD,), _f32),
        jax.ShapeDtypeStruct((_E,), _f32),
    ),
    mesh=_mesh,
    compiler_params=pltpu.CompilerParams(needs_layout_passes=False),
    scratch_types=(
        pltpu.VMEM((_EPT,), _i32),
        pltpu.VMEM((_EPT,), _f32),
        pltpu.VMEM((_EPT,), _f32),
        pltpu.VMEM((_NPAD,), _f32),
        pltpu.VMEM((_RPT,), _f32),
        pltpu.VMEM((_RPT,), _f32),
        pltpu.VMEM((_RPT,), _f32),
        pltpu.VMEM((_NPAD,), _f32),
        pltpu.VMEM_SHARED(((_NS + 1) * _NPAD,), _f32),
    ),
)(_k0_body)


_NCHT = _EPW // _CH
_NBUF = 4


def _conv_body(d, h_hbm, src_hbm, dst_hbm, w_hbm, out_hbm, *sc):
    sidx = sc[0:4]
    didx = sc[4:8]
    wc = sc[8:12]
    rows = sc[12:16]
    zrow = sc[16]
    acc_sp = sc[17]
    gs = sc[18:22]
    ss = sc[22:26]
    isem = sc[26:30]

    c = lax.axis_index("c")
    s_ = lax.axis_index("s")
    wid = c * _NS + s_
    base = wid * _EPW

    def _z(r, _):
        for cc in range(d // 16):
            zrow[r, pl.ds(cc * 16, 16)] = jnp.zeros((16,), _f32)
        return 0
    lax.fori_loop(0, 16, _z, 0)

    def _zc(k, _):
        pltpu.sync_copy(zrow, acc_sp.at[pl.ds(s_ * _RPT + k * 16, 16)])
        return 0
    lax.fori_loop(0, _RPT // 16, _zc, 0)

    plsc.subcore_barrier()

    def _swait(b):
        pltpu.make_async_copy(rows[b], acc_sp.at[didx[b]], ss[b]).wait()

    def _prep_idx(j, b):
        eb = base + j * _CH
        pltpu.async_copy(src_hbm.at[pl.ds(eb, _CH)], sidx[b], isem[b])
        pltpu.async_copy(dst_hbm.at[pl.ds(eb, _CH)], didx[b], isem[b])
        pltpu.async_copy(w_hbm.at[pl.ds(eb, _CH)], wc[b], isem[b])

    def _gissue(j, b):
        eb = base + j * _CH
        pltpu.make_async_copy(src_hbm.at[pl.ds(eb, _CH)], sidx[b], isem[b]).wait()
        pltpu.make_async_copy(dst_hbm.at[pl.ds(eb, _CH)], didx[b], isem[b]).wait()
        pltpu.make_async_copy(w_hbm.at[pl.ds(eb, _CH)], wc[b], isem[b]).wait()
        pltpu.async_copy(h_hbm.at[sidx[b]], rows[b], gs[b])

    def _process(b):
        pltpu.make_async_copy(h_hbm.at[sidx[b]], rows[b], gs[b]).wait()

        def _grp(g):
            w16 = wc[b][pl.ds(g * 16, 16)]
            for j in range(16):
                wj = _lane_splat(w16, j)
                e = g * 16 + j
                for cc in range(d // 16):
                    rows[b][e, pl.ds(cc * 16, 16)] = (
                        rows[b][e, pl.ds(cc * 16, 16)] * wj)
        plsc.parallel_loop(0, _CH // 16, unroll=5)(_grp)

        pltpu.async_copy(rows[b], acc_sp.at[didx[b]], ss[b], add=True)

    for j in range(3):
        _prep_idx(j, j)
    _gissue(0, 0)
    _gissue(1, 1)

    def _loop(t, _):
        for u in range(_NBUF):
            i = _NBUF * t + u
            _process(u)
            j2 = i + 2

            @pl.when(j2 < _NCHT)
            def _():
                _gissue(j2, (u + 2) % _NBUF)
            j3 = i + 3

            @pl.when(j3 < _NCHT)
            def _():
                b3 = (u + 3) % _NBUF

                @pl.when(j3 >= _NBUF)
                def _():
                    _swait(b3)
                _prep_idx(j3, b3)
        return 0
    lax.fori_loop(0, (_NCHT - 1) // _NBUF, _loop, 0)

    _process(0)
    for b in (1, 2, 3, 0):
        _swait(b)

    plsc.subcore_barrier()

    pltpu.sync_copy(acc_sp.at[pl.ds(s_ * _RPT, _RPT)],
                    out_hbm.at[pl.ds(c * _NPAD + s_ * _RPT, _RPT)])


def _make_conv(d):
    return functools.partial(
        pl.kernel,
        out_type=jax.ShapeDtypeStruct((_NC * _NPAD, d), _f32),
        mesh=_mesh,
        compiler_params=pltpu.CompilerParams(
            needs_layout_passes=False,
            use_tc_tiling_on_sc=(d == _D)),
        scratch_types=(
            [pltpu.VMEM((_CH,), _i32) for _ in range(4)]
            + [pltpu.VMEM((_CH,), _i32) for _ in range(4)]
            + [pltpu.VMEM((_CH,), _f32) for _ in range(4)]
            + [pltpu.VMEM((_CH, d), _f32) for _ in range(4)]
            + [pltpu.VMEM((16, d), _f32)]
            + [pltpu.VMEM_SHARED((_NPAD, d), _f32)]
            + [pltpu.SemaphoreType.DMA for _ in range(12)]
        ),
    )(functools.partial(_conv_body, d))


_conv128 = _make_conv(_D)
_conv32 = _make_conv(_DO)


def _lrelu(x):
    return jnp.where(x > 0, x, 0.01 * x)


def _gn_act(y, g, b, a):
    mean = jnp.sum(y, axis=0, keepdims=True) * (1.0 / _N)
    xc = y - a * mean
    var = jnp.sum(xc * xc, axis=0, keepdims=True) * (1.0 / _N)
    return _lrelu(g * xc * jax.lax.rsqrt(var + _EPS) + b)


def _t1_body(nf_ref, pp_ref, rin_ref, nw_ref, w_ref, g_ref, b_ref, a_ref,
             h_out, ro0_out, ro1_out):
    nw = nw_ref[...]
    ro0_out[...] = jnp.sum(nf_ref[...] * nw, axis=0, keepdims=True) * (1.0 / _N)
    pp = pp_ref[...]
    agg = (pp[0:_N, :] + pp[_NPAD:_NPAD + _N, :]) * rin_ref[...][0:_N, :]
    y = jnp.dot(agg, w_ref[...], preferred_element_type=_f32)
    h = _gn_act(y, g_ref[...], b_ref[...], a_ref[...])
    h_out[...] = h
    ro1_out[...] = jnp.sum(h * nw, axis=0, keepdims=True) * (1.0 / _N)


_t1 = pl.pallas_call(
    _t1_body,
    out_shape=(
        jax.ShapeDtypeStruct((_N, _D), _f32),
        jax.ShapeDtypeStruct((1, _D), _f32),
        jax.ShapeDtypeStruct((1, _D), _f32),
    ),
)


def _t2_body(pp_ref, rin_ref, nw_ref, w2_ref, g_ref, b_ref, a_ref, w3_ref,
             z_out, ro2_out):
    pp = pp_ref[...]
    agg = (pp[0:_N, :] + pp[_NPAD:_NPAD + _N, :]) * rin_ref[...][0:_N, :]
    y = jnp.dot(agg, w2_ref[...], preferred_element_type=_f32)
    h = _gn_act(y, g_ref[...], b_ref[...], a_ref[...])
    ro2_out[...] = jnp.sum(h * nw_ref[...], axis=0, keepdims=True) * (1.0 / _N)
    z_out[...] = jnp.dot(h, w3_ref[...], preferred_element_type=_f32)


_t2 = pl.pallas_call(
    _t2_body,
    out_shape=(
        jax.ShapeDtypeStruct((_N, _DO), _f32),
        jax.ShapeDtypeStruct((1, _D), _f32),
    ),
)


def _t3_body(pp_ref, rin_ref, nw_ref, g_ref, b_ref, a_ref,
             ro0_ref, ro1_ref, ro2_ref, out_ref):
    pp = pp_ref[...]
    y = (pp[0:_N, :] + pp[_NPAD:_NPAD + _N, :]) * rin_ref[...][0:_N, :]
    h = _gn_act(y, g_ref[...], b_ref[...], a_ref[...])
    ro3 = jnp.sum(h * nw_ref[...], axis=0, keepdims=True) * (1.0 / _N)
    out_ref[...] = _lrelu(
        jnp.concatenate([ro0_ref[...], ro1_ref[...], ro2_ref[...], ro3],
                        axis=1))


_t3 = pl.pallas_call(
    _t3_body,
    out_shape=jax.ShapeDtypeStruct((1, 3 * _D + _DO), _f32),
)


def kernel(node_feats, edge_index, edge_weights, node_weights,
           W1, W2, W3, g1, b1, a1, g2, b2, a2, g3, b3, a3):
    src = edge_index[0]
    dst = edge_index[1]

    r_in, w = _k0(src, dst, edge_weights)
    rin2 = r_in.reshape(_NPAD, 1)
    nw2 = node_weights.reshape(_N, 1)

    pp1 = _conv128(node_feats, src, dst, w)
    h1, ro0, ro1 = _t1(node_feats, pp1, rin2, nw2, W1,
                       g1.reshape(1, _D), b1.reshape(1, _D), a1.reshape(1, _D))

    pp2 = _conv128(h1, src, dst, w)
    z, ro2 = _t2(pp2, rin2, nw2, W2,
                 g2.reshape(1, _D), b2.reshape(1, _D), a2.reshape(1, _D), W3)

    pp3 = _conv32(z, src, dst, w)
    out = _t3(pp3, rin2, nw2,
              g3.reshape(1, _DO), b3.reshape(1, _DO), a3.reshape(1, _DO),
              ro0, ro1, ro2)
    return out


# speedup vs baseline: 1.2358x; 1.2358x over previous
import functools

import jax
import jax.numpy as jnp
from jax import lax
from jax.experimental import pallas as pl
from jax.experimental.pallas import tpu as pltpu
from jax.experimental.pallas import tpu_sc as plsc

_N = 10000
_NPAD = 10240
_E = 320000
_D = 128
_DO = 32
_EPS = 1e-5

_NC = 2
_NS = 16
_NW = _NC * _NS
_EPW = _E // _NW
_EPT = _E // _NS
_CH = 80
_NCHUNK = _EPW // _CH
_RPT = _NPAD // _NS

_mesh = plsc.VectorSubcoreMesh(
    core_axis_name="c", subcore_axis_name="s", num_cores=_NC, num_subcores=_NS)

_f32 = jnp.float32
_i32 = jnp.int32


def _rsqrt16(d):
    x = 0.5 * (d + 1.0)
    for _ in range(15):
        x = 0.5 * (x + d / x)
    return 1.0 / x


def _lane_splat(v16, j):
    return jax.lax.gather(
        v16, jnp.full((16, 1), j, _i32),
        jax.lax.GatherDimensionNumbers(
            offset_dims=(), collapsed_slice_dims=(0,), start_index_map=(0,)),
        slice_sizes=(1,),
        mode=jax.lax.GatherScatterMode.PROMISE_IN_BOUNDS)


def _k0_body(src_hbm, dst_hbm, ew_hbm, rin_hbm, w_hbm,
             eidx_res, ew_res, wbuf, ldeg, tbuf, dacc, rbuf, r_full, sh_sp):
    c = lax.axis_index("c")
    s = lax.axis_index("s")

    def _z(j, _):
        ldeg[pl.ds(j * 16, 16)] = jnp.zeros((16,), _f32)
        return 0
    lax.fori_loop(0, _NPAD // 16, _z, 0)

    @pl.when(c == 0)
    def _():
        pltpu.sync_copy(src_hbm.at[pl.ds(s * _EPT, _EPT)], eidx_res)

    @pl.when(c == 1)
    def _():
        pltpu.sync_copy(dst_hbm.at[pl.ds(s * _EPT, _EPT)], eidx_res)

    def _hist(i, _):
        idx16 = eidx_res[pl.ds(i * 16, 16)]
        plsc.addupdate_scatter(ldeg, [idx16], jnp.ones((16,), _f32))
        return 0
    lax.fori_loop(0, _EPT // 16, _hist, 0)

    pltpu.sync_copy(ldeg, sh_sp.at[pl.ds(s * _NPAD, _NPAD)])
    plsc.subcore_barrier()

    def _zt(j, _):
        dacc[pl.ds(j * 16, 16)] = jnp.zeros((16,), _f32)
        return 0
    lax.fori_loop(0, _RPT // 16, _zt, 0)

    def _red(r, _):
        pltpu.sync_copy(sh_sp.at[pl.ds(r * _NPAD + s * _RPT, _RPT)], tbuf)

        def _add(j, _):
            dacc[pl.ds(j * 16, 16)] = (dacc[pl.ds(j * 16, 16)]
                                       + tbuf[pl.ds(j * 16, 16)])
            return 0
        lax.fori_loop(0, _RPT // 16, _add, 0)
        return 0
    lax.fori_loop(0, _NS, _red, 0)

    def _r(j, _):
        d = jnp.maximum(dacc[pl.ds(j * 16, 16)], 1.0)
        rbuf[pl.ds(j * 16, 16)] = _rsqrt16(d)
        return 0
    lax.fori_loop(0, _RPT // 16, _r, 0)

    pltpu.sync_copy(rbuf, sh_sp.at[pl.ds(_NS * _NPAD + s * _RPT, _RPT)])

    @pl.when(c == 1)
    def _():
        pltpu.sync_copy(rbuf, rin_hbm.at[pl.ds(s * _RPT, _RPT)])

    plsc.subcore_barrier()

    @pl.when(c == 0)
    def _():
        pltpu.sync_copy(sh_sp.at[pl.ds(_NS * _NPAD, _NPAD)], r_full)
        pltpu.sync_copy(ew_hbm.at[pl.ds(s * _EPT, _EPT)], ew_res)

        def _w(i, _):
            idx16 = eidx_res[pl.ds(i * 16, 16)]
            r16 = plsc.load_gather(r_full, [idx16])
            wbuf[pl.ds(i * 16, 16)] = ew_res[pl.ds(i * 16, 16)] * r16
            return 0
        lax.fori_loop(0, _EPT // 16, _w, 0)
        pltpu.sync_copy(wbuf, w_hbm.at[pl.ds(s * _EPT, _EPT)])


_k0 = functools.partial(
    pl.kernel,
    out_type=(
        jax.ShapeDtypeStruct((_NPAD,), _f32),
        jax.ShapeDtypeStruct((_E,), _f32),
    ),
    mesh=_mesh,
    compiler_params=pltpu.CompilerParams(needs_layout_passes=False),
    scratch_types=(
        pltpu.VMEM((_EPT,), _i32),
        pltpu.VMEM((_EPT,), _f32),
        pltpu.VMEM((_EPT,), _f32),
        pltpu.VMEM((_NPAD,), _f32),
        pltpu.VMEM((_RPT,), _f32),
        pltpu.VMEM((_RPT,), _f32),
        pltpu.VMEM((_RPT,), _f32),
        pltpu.VMEM((_NPAD,), _f32),
        pltpu.VMEM_SHARED(((_NS + 1) * _NPAD,), _f32),
    ),
)(_k0_body)


_NCHT = _EPW // _CH
_NBUF = 4


def _conv_body(d, h_hbm, src_hbm, dst_hbm, w_hbm, out_hbm, *sc):
    sidx = sc[0:4]
    didx = sc[4:8]
    wc = sc[8:12]
    rows = sc[12:16]
    zrow = sc[16]
    acc_sp = sc[17]
    gs = sc[18:22]
    ss = sc[22:26]
    isem = sc[26:30]

    c = lax.axis_index("c")
    s_ = lax.axis_index("s")
    wid = c * _NS + s_
    base = wid * _EPW

    def _z(r, _):
        for cc in range(d // 16):
            zrow[r, pl.ds(cc * 16, 16)] = jnp.zeros((16,), _f32)
        return 0
    lax.fori_loop(0, 16, _z, 0)

    def _zc(k, _):
        pltpu.sync_copy(zrow, acc_sp.at[pl.ds(s_ * _RPT + k * 16, 16)])
        return 0
    lax.fori_loop(0, _RPT // 16, _zc, 0)

    plsc.subcore_barrier()

    def _swait(b):
        pltpu.make_async_copy(rows[b], acc_sp.at[didx[b]], ss[b]).wait()

    def _prep_idx(j, b):
        eb = base + j * _CH
        pltpu.async_copy(src_hbm.at[pl.ds(eb, _CH)], sidx[b], isem[b])
        pltpu.async_copy(dst_hbm.at[pl.ds(eb, _CH)], didx[b], isem[b])
        pltpu.async_copy(w_hbm.at[pl.ds(eb, _CH)], wc[b], isem[b])

    def _gissue(j, b):
        eb = base + j * _CH
        pltpu.make_async_copy(src_hbm.at[pl.ds(eb, _CH)], sidx[b], isem[b]).wait()
        pltpu.make_async_copy(dst_hbm.at[pl.ds(eb, _CH)], didx[b], isem[b]).wait()
        pltpu.make_async_copy(w_hbm.at[pl.ds(eb, _CH)], wc[b], isem[b]).wait()
        pltpu.async_copy(h_hbm.at[sidx[b]], rows[b], gs[b])

    def _process(b):
        pltpu.make_async_copy(h_hbm.at[sidx[b]], rows[b], gs[b]).wait()

        def _grp(g, _):
            w16 = wc[b][pl.ds(g * 16, 16)]
            for j in range(16):
                wj = _lane_splat(w16, j)
                e = g * 16 + j
                for cc in range(d // 16):
                    rows[b][e, pl.ds(cc * 16, 16)] = (
                        rows[b][e, pl.ds(cc * 16, 16)] * wj)
            return 0
        lax.fori_loop(0, _CH // 16, _grp, 0)

        pltpu.async_copy(rows[b], acc_sp.at[didx[b]], ss[b], add=True)

    for j in range(3):
        _prep_idx(j, j)
    _gissue(0, 0)
    _gissue(1, 1)

    def _loop(t, _):
        for u in range(_NBUF):
            i = _NBUF * t + u
            _process(u)
            j2 = i + 2

            @pl.when(j2 < _NCHT)
            def _():
                _gissue(j2, (u + 2) % _NBUF)
            j3 = i + 3

            @pl.when(j3 < _NCHT)
            def _():
                b3 = (u + 3) % _NBUF

                @pl.when(j3 >= _NBUF)
                def _():
                    _swait(b3)
                _prep_idx(j3, b3)
        return 0
    lax.fori_loop(0, (_NCHT - 1) // _NBUF, _loop, 0)

    _process(0)
    for b in (1, 2, 3, 0):
        _swait(b)

    plsc.subcore_barrier()

    pltpu.sync_copy(acc_sp.at[pl.ds(s_ * _RPT, _RPT)],
                    out_hbm.at[pl.ds(c * _NPAD + s_ * _RPT, _RPT)])


def _make_conv(d):
    return functools.partial(
        pl.kernel,
        out_type=jax.ShapeDtypeStruct((_NC * _NPAD, d), _f32),
        mesh=_mesh,
        compiler_params=pltpu.CompilerParams(
            needs_layout_passes=False,
            use_tc_tiling_on_sc=(d == _D)),
        scratch_types=(
            [pltpu.VMEM((_CH,), _i32) for _ in range(4)]
            + [pltpu.VMEM((_CH,), _i32) for _ in range(4)]
            + [pltpu.VMEM((_CH,), _f32) for _ in range(4)]
            + [pltpu.VMEM((_CH, d), _f32) for _ in range(4)]
            + [pltpu.VMEM((16, d), _f32)]
            + [pltpu.VMEM_SHARED((_NPAD, d), _f32)]
            + [pltpu.SemaphoreType.DMA for _ in range(12)]
        ),
    )(functools.partial(_conv_body, d))


_conv128 = _make_conv(_D)
_conv32 = _make_conv(_DO)


def _lrelu(x):
    return jnp.where(x > 0, x, 0.01 * x)


def _gn_act(y, g, b, a):
    mean = jnp.sum(y, axis=0, keepdims=True) * (1.0 / _N)
    xc = y - a * mean
    var = jnp.sum(xc * xc, axis=0, keepdims=True) * (1.0 / _N)
    return _lrelu(g * xc * jax.lax.rsqrt(var + _EPS) + b)


def _t1_body(nf_ref, pp_ref, rin_ref, nw_ref, w_ref, g_ref, b_ref, a_ref,
             h_out, ro0_out, ro1_out):
    nw = nw_ref[...]
    ro0_out[...] = jnp.sum(nf_ref[...] * nw, axis=0, keepdims=True) * (1.0 / _N)
    pp = pp_ref[...]
    agg = (pp[0:_N, :] + pp[_NPAD:_NPAD + _N, :]) * rin_ref[...][0:_N, :]
    y = jnp.dot(agg, w_ref[...], preferred_element_type=_f32)
    h = _gn_act(y, g_ref[...], b_ref[...], a_ref[...])
    h_out[...] = h
    ro1_out[...] = jnp.sum(h * nw, axis=0, keepdims=True) * (1.0 / _N)


_t1 = pl.pallas_call(
    _t1_body,
    out_shape=(
        jax.ShapeDtypeStruct((_N, _D), _f32),
        jax.ShapeDtypeStruct((1, _D), _f32),
        jax.ShapeDtypeStruct((1, _D), _f32),
    ),
)


def _t2_body(pp_ref, rin_ref, nw_ref, w2_ref, g_ref, b_ref, a_ref, w3_ref,
             z_out, ro2_out):
    pp = pp_ref[...]
    agg = (pp[0:_N, :] + pp[_NPAD:_NPAD + _N, :]) * rin_ref[...][0:_N, :]
    y = jnp.dot(agg, w2_ref[...], preferred_element_type=_f32)
    h = _gn_act(y, g_ref[...], b_ref[...], a_ref[...])
    ro2_out[...] = jnp.sum(h * nw_ref[...], axis=0, keepdims=True) * (1.0 / _N)
    z_out[...] = jnp.dot(h, w3_ref[...], preferred_element_type=_f32)


_t2 = pl.pallas_call(
    _t2_body,
    out_shape=(
        jax.ShapeDtypeStruct((_N, _DO), _f32),
        jax.ShapeDtypeStruct((1, _D), _f32),
    ),
)


def _t3_body(pp_ref, rin_ref, nw_ref, g_ref, b_ref, a_ref,
             ro0_ref, ro1_ref, ro2_ref, out_ref):
    pp = pp_ref[...]
    y = (pp[0:_N, :] + pp[_NPAD:_NPAD + _N, :]) * rin_ref[...][0:_N, :]
    h = _gn_act(y, g_ref[...], b_ref[...], a_ref[...])
    ro3 = jnp.sum(h * nw_ref[...], axis=0, keepdims=True) * (1.0 / _N)
    out_ref[...] = _lrelu(
        jnp.concatenate([ro0_ref[...], ro1_ref[...], ro2_ref[...], ro3],
                        axis=1))


_t3 = pl.pallas_call(
    _t3_body,
    out_shape=jax.ShapeDtypeStruct((1, 3 * _D + _DO), _f32),
)


def kernel(node_feats, edge_index, edge_weights, node_weights,
           W1, W2, W3, g1, b1, a1, g2, b2, a2, g3, b3, a3):
    src = edge_index[0]
    dst = edge_index[1]

    r_in, w = _k0(src, dst, edge_weights)
    rin2 = r_in.reshape(_NPAD, 1)
    nw2 = node_weights.reshape(_N, 1)

    pp1 = _conv128(node_feats, src, dst, w)
    h1, ro0, ro1 = _t1(node_feats, pp1, rin2, nw2, W1,
                       g1.reshape(1, _D), b1.reshape(1, _D), a1.reshape(1, _D))

    pp2 = _conv128(h1, src, dst, w)
    z, ro2 = _t2(pp2, rin2, nw2, W2,
                 g2.reshape(1, _D), b2.reshape(1, _D), a2.reshape(1, _D), W3)

    pp3 = _conv32(z, src, dst, w)
    out = _t3(pp3, rin2, nw2,
              g3.reshape(1, _DO), b3.reshape(1, _DO), a3.reshape(1, _DO),
              ro0, ro1, ro2)
    return out


# speedup vs baseline: 1.2709x; 1.0284x over previous
import functools

import jax
import jax.numpy as jnp
from jax import lax
from jax.experimental import pallas as pl
from jax.experimental.pallas import tpu as pltpu
from jax.experimental.pallas import tpu_sc as plsc

_N = 10000
_NPAD = 10240
_E = 320000
_D = 128
_DO = 32
_EPS = 1e-5

_NC = 2
_NS = 16
_NW = _NC * _NS
_EPW = _E // _NW
_EPT = _E // _NS
_CH = 80
_NCHUNK = _EPW // _CH
_RPT = _NPAD // _NS

_mesh = plsc.VectorSubcoreMesh(
    core_axis_name="c", subcore_axis_name="s", num_cores=_NC, num_subcores=_NS)

_f32 = jnp.float32
_i32 = jnp.int32


def _rsqrt16(d):
    x = 0.5 * (d + 1.0)
    for _ in range(15):
        x = 0.5 * (x + d / x)
    return 1.0 / x


def _lane_splat(v16, j):
    return jax.lax.gather(
        v16, jnp.full((16, 1), j, _i32),
        jax.lax.GatherDimensionNumbers(
            offset_dims=(), collapsed_slice_dims=(0,), start_index_map=(0,)),
        slice_sizes=(1,),
        mode=jax.lax.GatherScatterMode.PROMISE_IN_BOUNDS)


def _k0_body(src_hbm, dst_hbm, ew_hbm, rin_hbm, w_hbm,
             eidx_res, ew_res, wbuf, ldeg, tbuf, dacc, rbuf, r_full, sh_sp):
    c = lax.axis_index("c")
    s = lax.axis_index("s")

    def _z(j, _):
        ldeg[pl.ds(j * 16, 16)] = jnp.zeros((16,), _f32)
        return 0
    lax.fori_loop(0, _NPAD // 16, _z, 0)

    @pl.when(c == 0)
    def _():
        pltpu.sync_copy(src_hbm.at[pl.ds(s * _EPT, _EPT)], eidx_res)

    @pl.when(c == 1)
    def _():
        pltpu.sync_copy(dst_hbm.at[pl.ds(s * _EPT, _EPT)], eidx_res)

    def _hist(i, _):
        idx16 = eidx_res[pl.ds(i * 16, 16)]
        plsc.addupdate_scatter(ldeg, [idx16], jnp.ones((16,), _f32))
        return 0
    lax.fori_loop(0, _EPT // 16, _hist, 0)

    pltpu.sync_copy(ldeg, sh_sp.at[pl.ds(s * _NPAD, _NPAD)])
    plsc.subcore_barrier()

    def _zt(j, _):
        dacc[pl.ds(j * 16, 16)] = jnp.zeros((16,), _f32)
        return 0
    lax.fori_loop(0, _RPT // 16, _zt, 0)

    def _red(r, _):
        pltpu.sync_copy(sh_sp.at[pl.ds(r * _NPAD + s * _RPT, _RPT)], tbuf)

        def _add(j, _):
            dacc[pl.ds(j * 16, 16)] = (dacc[pl.ds(j * 16, 16)]
                                       + tbuf[pl.ds(j * 16, 16)])
            return 0
        lax.fori_loop(0, _RPT // 16, _add, 0)
        return 0
    lax.fori_loop(0, _NS, _red, 0)

    def _r(j, _):
        d = jnp.maximum(dacc[pl.ds(j * 16, 16)], 1.0)
        rbuf[pl.ds(j * 16, 16)] = _rsqrt16(d)
        return 0
    lax.fori_loop(0, _RPT // 16, _r, 0)

    pltpu.sync_copy(rbuf, sh_sp.at[pl.ds(_NS * _NPAD + s * _RPT, _RPT)])

    @pl.when(c == 1)
    def _():
        pltpu.sync_copy(rbuf, rin_hbm.at[pl.ds(s * _RPT, _RPT)])

    plsc.subcore_barrier()

    @pl.when(c == 0)
    def _():
        pltpu.sync_copy(sh_sp.at[pl.ds(_NS * _NPAD, _NPAD)], r_full)
        pltpu.sync_copy(ew_hbm.at[pl.ds(s * _EPT, _EPT)], ew_res)

        def _w(i, _):
            idx16 = eidx_res[pl.ds(i * 16, 16)]
            r16 = plsc.load_gather(r_full, [idx16])
            wbuf[pl.ds(i * 16, 16)] = ew_res[pl.ds(i * 16, 16)] * r16
            return 0
        lax.fori_loop(0, _EPT // 16, _w, 0)
        pltpu.sync_copy(wbuf, w_hbm.at[pl.ds(s * _EPT, _EPT)])


_k0 = functools.partial(
    pl.kernel,
    out_type=(
        jax.ShapeDtypeStruct((_NPAD,), _f32),
        jax.ShapeDtypeStruct((_E,), _f32),
    ),
    mesh=_mesh,
    compiler_params=pltpu.CompilerParams(needs_layout_passes=False),
    scratch_types=(
        pltpu.VMEM((_EPT,), _i32),
        pltpu.VMEM((_EPT,), _f32),
        pltpu.VMEM((_EPT,), _f32),
        pltpu.VMEM((_NPAD,), _f32),
        pltpu.VMEM((_RPT,), _f32),
        pltpu.VMEM((_RPT,), _f32),
        pltpu.VMEM((_RPT,), _f32),
        pltpu.VMEM((_NPAD,), _f32),
        pltpu.VMEM_SHARED(((_NS + 1) * _NPAD,), _f32),
    ),
)(_k0_body)


_NCHT = _EPW // _CH
_NBUF = 4


def _conv_body(d, h_hbm, src_hbm, dst_hbm, w_hbm, out_hbm, *sc):
    sidx = sc[0:4]
    didx = sc[4:8]
    wc = sc[8:12]
    rows = sc[12:16]
    zrow = sc[16]
    acc_sp = sc[17]
    gs = sc[18:22]
    ss = sc[22:26]
    isem = sc[26:30]

    c = lax.axis_index("c")
    s_ = lax.axis_index("s")
    wid = c * _NS + s_
    base = wid * _EPW

    def _z(r, _):
        for cc in range(d // 16):
            zrow[r, pl.ds(cc * 16, 16)] = jnp.zeros((16,), _f32)
        return 0
    lax.fori_loop(0, 16, _z, 0)

    def _zc(k, _):
        pltpu.sync_copy(zrow, acc_sp.at[pl.ds(s_ * _RPT + k * 16, 16)])
        return 0
    lax.fori_loop(0, _RPT // 16, _zc, 0)

    plsc.subcore_barrier()

    def _swait(b):
        pltpu.make_async_copy(rows[b], acc_sp.at[didx[b]], ss[b]).wait()

    def _prep_idx(j, b):
        eb = base + j * _CH
        pltpu.async_copy(src_hbm.at[pl.ds(eb, _CH)], sidx[b], isem[b])
        pltpu.async_copy(dst_hbm.at[pl.ds(eb, _CH)], didx[b], isem[b])
        pltpu.async_copy(w_hbm.at[pl.ds(eb, _CH)], wc[b], isem[b])

    def _gissue(j, b):
        eb = base + j * _CH
        pltpu.make_async_copy(src_hbm.at[pl.ds(eb, _CH)], sidx[b], isem[b]).wait()
        pltpu.make_async_copy(dst_hbm.at[pl.ds(eb, _CH)], didx[b], isem[b]).wait()
        pltpu.make_async_copy(w_hbm.at[pl.ds(eb, _CH)], wc[b], isem[b]).wait()
        pltpu.async_copy(h_hbm.at[sidx[b]], rows[b], gs[b])

    def _process(b):
        pltpu.make_async_copy(h_hbm.at[sidx[b]], rows[b], gs[b]).wait()

        def _grp(g, _):
            w16 = wc[b][pl.ds(g * 16, 16)]
            for j in range(16):
                wj = _lane_splat(w16, j)
                e = g * 16 + j
                for cc in range(d // 16):
                    rows[b][e, pl.ds(cc * 16, 16)] = (
                        rows[b][e, pl.ds(cc * 16, 16)] * wj)
            return 0
        lax.fori_loop(0, _CH // 16, _grp, 0)

        pltpu.async_copy(rows[b], acc_sp.at[didx[b]], ss[b], add=True)

    for j in range(3):
        _prep_idx(j, j)
    _gissue(0, 0)
    _gissue(1, 1)

    def _loop(t, _):
        for u in range(_NBUF):
            i = _NBUF * t + u
            _process(u)
            j2 = i + 2

            @pl.when(j2 < _NCHT)
            def _():
                _gissue(j2, (u + 2) % _NBUF)
            j3 = i + 3

            @pl.when(j3 < _NCHT)
            def _():
                b3 = (u + 3) % _NBUF

                @pl.when(j3 >= _NBUF)
                def _():
                    _swait(b3)
                _prep_idx(j3, b3)
        return 0
    lax.fori_loop(0, (_NCHT - 1) // _NBUF, _loop, 0)

    _process(0)
    for b in (1, 2, 3, 0):
        _swait(b)

    plsc.subcore_barrier()

    pltpu.sync_copy(acc_sp.at[pl.ds(s_ * _RPT, _RPT)],
                    out_hbm.at[pl.ds(c * _NPAD + s_ * _RPT, _RPT)])


def _make_conv(d):
    return functools.partial(
        pl.kernel,
        out_type=jax.ShapeDtypeStruct((_NC * _NPAD, d), _f32),
        mesh=_mesh,
        compiler_params=pltpu.CompilerParams(
            needs_layout_passes=False,
            use_tc_tiling_on_sc=(d == _D)),
        scratch_types=(
            [pltpu.VMEM((_CH,), _i32) for _ in range(4)]
            + [pltpu.VMEM((_CH,), _i32) for _ in range(4)]
            + [pltpu.VMEM((_CH,), _f32) for _ in range(4)]
            + [pltpu.VMEM((_CH, d), _f32) for _ in range(4)]
            + [pltpu.VMEM((16, d), _f32)]
            + [pltpu.VMEM_SHARED((_NPAD, d), _f32)]
            + [pltpu.SemaphoreType.DMA for _ in range(12)]
        ),
    )(functools.partial(_conv_body, d))


_conv128 = _make_conv(_D)

_CHV = 128
_NCHG = _E // _CHV


def _conv32_body(h_hbm, ed_hbm, zer_hbm, out_hbm, *sc):
    d = _DO
    ebuf = sc[0:4]
    rows = sc[4:8]
    acc_sp = sc[8]
    gs = sc[9:13]
    ss = sc[13:17]
    isem = sc[17:21]

    c = lax.axis_index("c")
    s_ = lax.axis_index("s")
    wid = c * _NS + s_
    glo = (wid * _NCHG) // _NW
    ncht = ((wid + 1) * _NCHG) // _NW - glo

    @pl.when(s_ < 15)
    def _():
        pltpu.sync_copy(zer_hbm.at[pl.ds(s_ * 624, 624)],
                        acc_sp.at[pl.ds(s_ * 624, 624)])

    @pl.when(s_ == 15)
    def _():
        pltpu.sync_copy(zer_hbm.at[pl.ds(9360, 640)],
                        acc_sp.at[pl.ds(9360, 640)])

    plsc.subcore_barrier()

    def _prep_idx(j, b):
        pltpu.async_copy(ed_hbm.at[glo + j], ebuf[b], isem[b])

    def _swait(b):
        pltpu.make_async_copy(rows[b], acc_sp.at[ebuf[b].at[1]], ss[b]).wait()

    def _gissue(j, b):
        pltpu.make_async_copy(ed_hbm.at[glo + j], ebuf[b], isem[b]).wait()
        pltpu.async_copy(h_hbm.at[ebuf[b].at[0]], rows[b], gs[b])

    def _process(b):
        pltpu.make_async_copy(h_hbm.at[ebuf[b].at[0]], rows[b], gs[b]).wait()

        def _grp(g, _):
            w16 = plsc.bitcast(ebuf[b][2, pl.ds(g * 16, 16)], _f32)
            for j in range(16):
                wj = _lane_splat(w16, j)
                e = g * 16 + j
                for cc in range(d // 16):
                    rows[b][e, pl.ds(cc * 16, 16)] = (
                        rows[b][e, pl.ds(cc * 16, 16)] * wj)
            return 0
        lax.fori_loop(0, _CHV // 16, _grp, 0)

        pltpu.async_copy(rows[b], acc_sp.at[ebuf[b].at[1]], ss[b], add=True)

    for j in range(3):
        _prep_idx(j, j)
    _gissue(0, 0)
    _gissue(1, 1)

    def _loop(t, _):
        for u in range(4):
            i = 4 * t + u

            @pl.when(i < ncht)
            def _():
                _process(u)
            j2 = i + 2

            @pl.when(j2 < ncht)
            def _():
                _gissue(j2, (u + 2) % 4)
            j3 = i + 3

            @pl.when(j3 < ncht)
            def _():
                b3 = (u + 3) % 4

                @pl.when(j3 >= 4)
                def _():
                    _swait(b3)
                _prep_idx(j3, b3)
        return 0
    lax.fori_loop(0, 20, _loop, 0)

    for b in range(4):
        _swait(b)

    plsc.subcore_barrier()

    @pl.when(s_ < 15)
    def _():
        pltpu.sync_copy(acc_sp.at[pl.ds(s_ * 624, 624)],
                        out_hbm.at[pl.ds(c * _N + s_ * 624, 624)])

    @pl.when(s_ == 15)
    def _():
        pltpu.sync_copy(acc_sp.at[pl.ds(9360, 640)],
                        out_hbm.at[pl.ds(c * _N + 9360, 640)])


_conv32 = functools.partial(
    pl.kernel,
    out_type=jax.ShapeDtypeStruct((_NC * _N, _DO), _f32),
    mesh=_mesh,
    compiler_params=pltpu.CompilerParams(
        needs_layout_passes=False, use_tc_tiling_on_sc=False),
    scratch_types=(
        [pltpu.VMEM((3, _CHV), _i32) for _ in range(4)]
        + [pltpu.VMEM((_CHV, _DO), _f32) for _ in range(4)]
        + [pltpu.VMEM_SHARED((_N, _DO), _f32)]
        + [pltpu.SemaphoreType.DMA for _ in range(12)]
    ),
)(_conv32_body)


def _lrelu(x):
    return jnp.where(x > 0, x, 0.01 * x)


def _gn_act(y, g, b, a):
    mean = jnp.sum(y, axis=0, keepdims=True) * (1.0 / _N)
    xc = y - a * mean
    var = jnp.sum(xc * xc, axis=0, keepdims=True) * (1.0 / _N)
    return _lrelu(g * xc * jax.lax.rsqrt(var + _EPS) + b)


def _t1_body(nf_ref, pp_ref, rin_ref, nw_ref, w_ref, g_ref, b_ref, a_ref,
             h_out, ro0_out, ro1_out):
    nw = nw_ref[...]
    ro0_out[...] = jnp.sum(nf_ref[...] * nw, axis=0, keepdims=True) * (1.0 / _N)
    pp = pp_ref[...]
    agg = (pp[0:_N, :] + pp[_NPAD:_NPAD + _N, :]) * rin_ref[...][0:_N, :]
    y = jnp.dot(agg, w_ref[...], preferred_element_type=_f32)
    h = _gn_act(y, g_ref[...], b_ref[...], a_ref[...])
    h_out[...] = h
    ro1_out[...] = jnp.sum(h * nw, axis=0, keepdims=True) * (1.0 / _N)


_t1 = pl.pallas_call(
    _t1_body,
    out_shape=(
        jax.ShapeDtypeStruct((_N, _D), _f32),
        jax.ShapeDtypeStruct((1, _D), _f32),
        jax.ShapeDtypeStruct((1, _D), _f32),
    ),
)


def _t2_body(pp_ref, rin_ref, nw_ref, w2_ref, g_ref, b_ref, a_ref, w3_ref,
             z_out, ro2_out):
    pp = pp_ref[...]
    agg = (pp[0:_N, :] + pp[_NPAD:_NPAD + _N, :]) * rin_ref[...][0:_N, :]
    y = jnp.dot(agg, w2_ref[...], preferred_element_type=_f32)
    h = _gn_act(y, g_ref[...], b_ref[...], a_ref[...])
    ro2_out[...] = jnp.sum(h * nw_ref[...], axis=0, keepdims=True) * (1.0 / _N)
    z_out[...] = jnp.dot(h, w3_ref[...], preferred_element_type=_f32)


_t2 = pl.pallas_call(
    _t2_body,
    out_shape=(
        jax.ShapeDtypeStruct((_N, _DO), _f32),
        jax.ShapeDtypeStruct((1, _D), _f32),
    ),
)


def _t3_body(pp_ref, rin_ref, nw_ref, g_ref, b_ref, a_ref,
             ro0_ref, ro1_ref, ro2_ref, out_ref):
    pp = pp_ref[...]
    y = (pp[0:_N, :] + pp[_N:2 * _N, :]) * rin_ref[...][0:_N, :]
    h = _gn_act(y, g_ref[...], b_ref[...], a_ref[...])
    ro3 = jnp.sum(h * nw_ref[...], axis=0, keepdims=True) * (1.0 / _N)
    out_ref[...] = _lrelu(
        jnp.concatenate([ro0_ref[...], ro1_ref[...], ro2_ref[...], ro3],
                        axis=1))


_t3 = pl.pallas_call(
    _t3_body,
    out_shape=jax.ShapeDtypeStruct((1, 3 * _D + _DO), _f32),
)


def kernel(node_feats, edge_index, edge_weights, node_weights,
           W1, W2, W3, g1, b1, a1, g2, b2, a2, g3, b3, a3):
    src = edge_index[0]
    dst = edge_index[1]

    r_in, w = _k0(src, dst, edge_weights)
    rin2 = r_in.reshape(_NPAD, 1)
    nw2 = node_weights.reshape(_N, 1)

    pp1 = _conv128(node_feats, src, dst, w)
    h1, ro0, ro1 = _t1(node_feats, pp1, rin2, nw2, W1,
                       g1.reshape(1, _D), b1.reshape(1, _D), a1.reshape(1, _D))

    pp2 = _conv128(h1, src, dst, w)
    z, ro2 = _t2(pp2, rin2, nw2, W2,
                 g2.reshape(1, _D), b2.reshape(1, _D), a2.reshape(1, _D), W3)

    wbits = jax.lax.bitcast_convert_type(w, _i32)
    ed = jnp.stack([src.reshape(_NCHG, _CHV), dst.reshape(_NCHG, _CHV),
                    wbits.reshape(_NCHG, _CHV)], axis=1)
    pp3 = _conv32(z, ed, jnp.zeros((_N, _DO), _f32))
    out = _t3(pp3, rin2, nw2,
              g3.reshape(1, _DO), b3.reshape(1, _DO), a3.reshape(1, _DO),
              ro0, ro1, ro2)
    return out
